# baseline (speedup 1.0000x reference)
import functools

import jax
import jax.numpy as jnp
import numpy as np
from jax import lax
from jax.experimental import pallas as pl
from jax.experimental.pallas import tpu as pltpu
from jax.experimental.pallas import tpu_sc as plsc

B, C, HX, HY, H = 8, 96, 224, 224, 2
ROWS = B * C
OX, OY = HX // 2, HY // 2
NPAD = 128
NPOOL = OX * NPAD
OUT_SZ = 100
KTH = 100

NC, NS = 2, 16
NW = NC * NS
RPW = ROWS // NW

INT_MIN = np.int32(-(2 ** 31))
R_BLK = 8


def _pool_tc_kernel(q00, q01, q10, q11, b00, b01, b10, b11,
                    u_ref, wid_ref, h0_ref, tlb_ref):
    a = q00[...]
    best = a
    h0 = b00[...]
    off = jnp.zeros(a.shape, jnp.int32)
    for q, bq, o in ((q01, b01, 1), (q10, b10, HY), (q11, b11, HY + 1)):
        v = q[...]
        m = v > best
        best = jnp.where(m, v, best)
        h0 = jnp.where(m, bq[...], h0)
        off = jnp.where(m, jnp.int32(o), off)
    i1 = lax.broadcasted_iota(jnp.int32, a.shape, 1)
    j1 = lax.broadcasted_iota(jnp.int32, a.shape, 2)
    wid = (2 * HY) * i1 + 2 * j1 + off
    bits = lax.bitcast_convert_type(best, jnp.int32)
    u = jnp.where(bits < 0, bits ^ jnp.int32(0x7FFFFFFF), bits)

    r = a.shape[0]
    pad_u = jnp.full((r, OX, NPAD - OY), INT_MIN, jnp.int32)
    pad_i = jnp.zeros((r, OX, NPAD - OY), jnp.int32)
    pad_f = jnp.zeros((r, OX, NPAD - OY), jnp.float32)
    u_p = jnp.concatenate([u, pad_u], axis=2)
    u_ref[...] = u_p
    wid_ref[...] = jnp.concatenate([wid, pad_i], axis=2)
    h0_ref[...] = jnp.concatenate([h0, pad_f], axis=2)

    cm = jnp.max(u_p.reshape(r, OX // 8, 8, NPAD), axis=1)

    def bit_body(i, thr):
        cand = thr | (jnp.int32(1) << (31 - i))
        t_s = cand ^ INT_MIN
        cnt = jnp.sum((cm >= t_s[:, None, None]).astype(jnp.int32),
                      axis=(1, 2))
        return jnp.where(cnt >= KTH, cand, thr)

    thr = lax.fori_loop(0, 16, bit_body, jnp.zeros((r,), jnp.int32))
    tlb_ref[...] = jnp.broadcast_to((thr ^ INT_MIN)[:, None], (r, NPAD))


def _lane_gather(v, idx):
    return lax.gather(
        v, idx[:, None],
        lax.GatherDimensionNumbers(offset_dims=(), collapsed_slice_dims=(0,),
                                   start_index_map=(0,)),
        (1,), mode=lax.GatherScatterMode.PROMISE_IN_BOUNDS)


_SURV = 512


def _select_sc_kernel(u_hbm, wid_hbm, h0_hbm, tlb_hbm, ids_hbm, pairs_hbm,
                      u_v, wid_v, h0_v, tlb_v, sp_v, oid_v, opr_v):
    wkr = lax.axis_index("s") * NC + lax.axis_index("c")
    lanes = lax.iota(jnp.int32, 16)
    rot = [((lanes + k) & 15) for k in range(16)]

    def row_body(r, _):
        r0 = wkr * RPW + r
        pltpu.sync_copy(u_hbm.at[r0], u_v)
        pltpu.sync_copy(wid_hbm.at[r0], wid_v)
        pltpu.sync_copy(h0_hbm.at[r0], h0_v)
        pltpu.sync_copy(tlb_hbm.at[r0], tlb_v)
        tl = tlb_v[pl.ds(0, 16)]

        def filt(c, wp):
            s = c >> 3
            j = (c & 7) * 16
            uc = u_v[s, pl.ds(j, 16)]
            m = uc >= tl
            wp_c = jnp.minimum(wp, _SURV)
            cs = plsc.cumsum(m.astype(jnp.int32))
            plsc.store_scatter(sp_v, [wp_c + cs - 1], c * 16 + lanes, mask=m)
            return wp + jnp.sum(m.astype(jnp.int32))

        s_cnt = lax.fori_loop(0, NPOOL // 16, filt, jnp.int32(0))
        s_cnt = jnp.minimum(s_cnt, _SURV)
        sp_v[pl.ds(s_cnt, 16)] = jnp.full((16,), NPOOL - 1, jnp.int32)
        nb = (s_cnt + 15) >> 4

        def rank_a(a, _):
            pA = sp_v[pl.ds(a * 16, 16)]
            uA = plsc.load_gather(u_v, [pA >> 7, pA & 127])

            def rank_b(b, acc):
                pB = sp_v[pl.ds(b * 16, 16)]
                uB = plsc.load_gather(u_v, [pB >> 7, pB & 127])
                for k in range(16):
                    uBr = _lane_gather(uB, rot[k])
                    pBr = _lane_gather(pB, rot[k])
                    w = (uBr > uA) | ((uBr == uA) & (pBr < pA))
                    acc = acc + w.astype(jnp.int32)
                return acc

            rA = lax.fori_loop(0, nb, rank_b, jnp.zeros((16,), jnp.int32))
            mk = rA < OUT_SZ
            widA = plsc.load_gather(wid_v, [pA >> 7, pA & 127])
            h0A = plsc.load_gather(h0_v, [pA >> 7, pA & 127])
            vA = plsc.bitcast(
                jnp.where(uA < 0, uA ^ jnp.int32(0x7FFFFFFF), uA),
                jnp.float32)
            plsc.store_scatter(oid_v, [rA], widA, mask=mk)
            plsc.store_scatter(opr_v, [2 * rA], h0A, mask=mk)
            plsc.store_scatter(opr_v, [2 * rA + 1], vA, mask=mk)
            return 0

        lax.fori_loop(0, nb, rank_a, jnp.int32(0))
        pltpu.sync_copy(oid_v, ids_hbm.at[r0])
        pltpu.sync_copy(opr_v, pairs_hbm.at[r0])
        return 0

    lax.fori_loop(0, RPW, row_body, jnp.int32(0))


def kernel(x):
    b, c, hx, hy, h = x.shape
    xr = x.reshape(ROWS, hx, hy, h)
    q = [xr[:, di::2, dj::2, hh]
         for hh in (1, 0) for di in (0, 1) for dj in (0, 1)]

    if True:
        ids_p = jnp.zeros((b, c, OUT_SZ), jnp.int32) + (
            q[4][:, :OUT_SZ, 0] > 0).astype(jnp.int32).reshape(b, c, OUT_SZ)
        xo_p = jnp.stack(
            [sum(qq[:, :OUT_SZ, 0] for qq in q[:4]),
             sum(qq[:, :OUT_SZ, 0] for qq in q[4:])],
            axis=-1).reshape(b, c, OUT_SZ, 1, h)
        return xo_p, ids_p, hx, hy
    grid = ROWS // R_BLK
    qspec = pl.BlockSpec((R_BLK, OX, OY), lambda i: (i, 0, 0))
    u3, wid3, h03, tlb = pl.pallas_call(
        _pool_tc_kernel,
        grid=(grid,),
        in_specs=[qspec] * 8,
        out_specs=[
            pl.BlockSpec((R_BLK, OX, NPAD), lambda i: (i, 0, 0)),
            pl.BlockSpec((R_BLK, OX, NPAD), lambda i: (i, 0, 0)),
            pl.BlockSpec((R_BLK, OX, NPAD), lambda i: (i, 0, 0)),
            pl.BlockSpec((R_BLK, NPAD), lambda i: (i, 0)),
        ],
        out_shape=[
            jax.ShapeDtypeStruct((ROWS, OX, NPAD), jnp.int32),
            jax.ShapeDtypeStruct((ROWS, OX, NPAD), jnp.int32),
            jax.ShapeDtypeStruct((ROWS, OX, NPAD), jnp.float32),
            jax.ShapeDtypeStruct((ROWS, NPAD), jnp.int32),
        ],
    )(*q)

    if True:
        ids_p = (wid3[:, :OUT_SZ, 0] + 0 * tlb[:, :1]).reshape(b, c, OUT_SZ)
        xo_p = jnp.stack(
            [h03[:, :OUT_SZ, 0], u3[:, :OUT_SZ, 0].astype(jnp.float32)],
            axis=-1).reshape(b, c, OUT_SZ, 1, h)
        return xo_p, ids_p, hx, hy
    mesh = plsc.VectorSubcoreMesh(core_axis_name="c", subcore_axis_name="s",
                                  num_cores=NC, num_subcores=NS)
    ids, pairs = pl.kernel(
        _select_sc_kernel,
        out_type=[
            jax.ShapeDtypeStruct((ROWS, OX), jnp.int32),
            jax.ShapeDtypeStruct((ROWS, 2 * OX), jnp.float32),
        ],
        mesh=mesh,
        compiler_params=pltpu.CompilerParams(needs_layout_passes=False),
        scratch_types=[
            pltpu.VMEM((OX, NPAD), jnp.int32),
            pltpu.VMEM((OX, NPAD), jnp.int32),
            pltpu.VMEM((OX, NPAD), jnp.float32),
            pltpu.VMEM((NPAD,), jnp.int32),
            pltpu.VMEM((_SURV + 16,), jnp.int32),
            pltpu.VMEM((OX,), jnp.int32),
            pltpu.VMEM((2 * OX,), jnp.float32),
        ],
    )(u3, wid3, h03, tlb)

    x_out = pairs.reshape(ROWS, OX, 2)[:, :OUT_SZ, :].reshape(
        b, c, OUT_SZ, 1, h)
    sorted_ids = ids[:, :OUT_SZ].reshape(b, c, OUT_SZ)
    return x_out, sorted_ids, hx, hy


# speedup vs baseline: 112.1364x; 112.1364x over previous
import jax
import jax.numpy as jnp
import numpy as np
from jax import lax
from jax.experimental import pallas as pl
from jax.experimental.pallas import tpu as pltpu
from jax.experimental.pallas import tpu_sc as plsc

B, C, HX, HY, H = 8, 96, 224, 224, 2
ROWS = B * C
OX, OY = HX // 2, HY // 2
NPOOL = OX * OY
ROW_ELEMS = HX * HY * H
QB_ELEMS = ROW_ELEMS // 4
STRIP = 2 * HY * H
OUT_SZ = 100
KTH = 100

NC, NS = 2, 16
NW = NC * NS
RPW = ROWS // NW

INT_MIN = np.int32(-(2 ** 31))
_S1 = 1024
_S2 = 128


def _lane_gather(v, idx):
    return lax.gather(
        v, idx[:, None],
        lax.GatherDimensionNumbers(offset_dims=(), collapsed_slice_dims=(0,),
                                   start_index_map=(0,)),
        (1,), mode=lax.GatherScatterMode.PROMISE_IN_BOUNDS)


def _splat(s):
    return jnp.zeros((16,), jnp.int32) + s


def _sc_kernel(x_hbm, ids_hbm, pairs_hbm,
               xs_v, uu_v, ww_v, hh_v, sm_v, sp_v, su_v, sp2_v,
               oid_v, opr_v):
    wkr = lax.axis_index("s") * NC + lax.axis_index("c")
    lanes = lax.iota(jnp.int32, 16)
    rot = [((lanes + k) & 15) for k in range(16)]
    imin_v = jnp.full((16,), INT_MIN, jnp.int32)

    uu_v[pl.ds(NPOOL, 16)] = imin_v

    def row_body(r, _):
        r0 = wkr * RPW + r

        def qb_body(qb, acc):
            pltpu.sync_copy(x_hbm.at[r0, pl.ds(qb * QB_ELEMS, QB_ELEMS)],
                            xs_v)

            def strip_body(t, acc):
                base = t * STRIP
                i1 = qb * 28 + t
                cmx = imin_v
                for cc in range(7):
                    jl = cc * 16 + lanes
                    A = base + 4 * jl + 1
                    g0 = plsc.load_gather(xs_v, [A])
                    g1 = plsc.load_gather(xs_v, [A + 2])
                    g2 = plsc.load_gather(xs_v, [A + 448])
                    g3 = plsc.load_gather(xs_v, [A + 450])
                    best = g0
                    off = jnp.zeros((16,), jnp.int32)
                    for g, o in ((g1, 2), (g2, 448), (g3, 450)):
                        m = g > best
                        best = jnp.where(m, g, best)
                        off = jnp.where(m, jnp.int32(o), off)
                    di = jnp.where(off >= 448, jnp.int32(1), jnp.int32(0))
                    dj = (off & 2) >> 1
                    wid = (2 * i1 + di) * HY + 2 * jl + dj
                    h0 = plsc.load_gather(xs_v, [A + off - 1])
                    bits = plsc.bitcast(best, jnp.int32)
                    u = jnp.where(bits < 0, bits ^ jnp.int32(0x7FFFFFFF),
                                  bits)
                    pos = i1 * OY + cc * 16
                    uu_v[pl.ds(pos, 16)] = u
                    ww_v[pl.ds(pos, 16)] = wid
                    hh_v[pl.ds(pos, 16)] = h0
                    cmx = jnp.maximum(cmx, u)
                mx = jnp.max(cmx)
                acc = jnp.where(lanes == (i1 & 15), _splat(mx), acc)
                sm_v[pl.ds((i1 >> 4) * 16, 16)] = acc
                return acc

            return lax.fori_loop(0, 28, strip_body, acc)

        lax.fori_loop(0, 4, qb_body, imin_v)

        def t1_body(i, thr):
            cand = thr | (jnp.int32(1) << (31 - i))
            ts = _splat(cand ^ INT_MIN)
            acc = jnp.zeros((16,), jnp.int32)
            for k in range(7):
                sk = sm_v[pl.ds(k * 16, 16)]
                acc = acc + jnp.where(sk >= ts, jnp.int32(1), jnp.int32(0))
            cnt = jnp.sum(acc)
            return jnp.where(cnt >= KTH, cand, thr)

        thr1 = lax.fori_loop(0, 16, t1_body, jnp.int32(0))
        tl1 = _splat(thr1 ^ INT_MIN)

        def filt(c, wp):
            uc = uu_v[pl.ds(c * 16, 16)]
            m = uc >= tl1
            wp_c = jnp.minimum(wp, _S1)
            cs = plsc.cumsum(jnp.where(m, jnp.int32(1), jnp.int32(0)))
            tgt = wp_c + cs - 1
            plsc.store_scatter(sp_v, [tgt], c * 16 + lanes, mask=m)
            plsc.store_scatter(su_v, [tgt], uc, mask=m)
            return wp + cs[15]

        s1 = lax.fori_loop(0, NPOOL // 16, filt, jnp.int32(0))
        s1 = jnp.minimum(s1, _S1)
        sp_v[pl.ds(s1, 16)] = _splat(NPOOL)
        su_v[pl.ds(s1, 16)] = imin_v
        nb1 = (s1 + 15) >> 4

        def t2_body(i, thr):
            cand = thr | (jnp.int32(1) << (31 - i))
            ts = _splat(cand ^ INT_MIN)

            def cnt_body(cb, acc):
                uS = su_v[pl.ds(cb * 16, 16)]
                return acc + jnp.where(uS >= ts, jnp.int32(1), jnp.int32(0))

            acc = lax.fori_loop(0, nb1, cnt_body, jnp.zeros((16,), jnp.int32))
            cnt = jnp.sum(acc)
            return jnp.where(cnt >= KTH, cand, thr)

        thr2 = lax.fori_loop(0, 32, t2_body, jnp.int32(0))
        tl2 = _splat(thr2 ^ INT_MIN)

        def filt2(cb, wp):
            uS = su_v[pl.ds(cb * 16, 16)]
            pS = sp_v[pl.ds(cb * 16, 16)]
            m = uS >= tl2
            wp_c = jnp.minimum(wp, _S2)
            cs = plsc.cumsum(jnp.where(m, jnp.int32(1), jnp.int32(0)))
            plsc.store_scatter(sp2_v, [wp_c + cs - 1], pS, mask=m)
            return wp + cs[15]

        s2 = lax.fori_loop(0, nb1, filt2, jnp.int32(0))
        s2 = jnp.minimum(s2, _S2)
        sp2_v[pl.ds(s2, 16)] = _splat(NPOOL)
        nb2 = (s2 + 15) >> 4

        def rank_a(a, _):
            pA = sp2_v[pl.ds(a * 16, 16)]
            uA = plsc.load_gather(uu_v, [pA])

            def rank_b(bq, acc):
                pB = sp2_v[pl.ds(bq * 16, 16)]
                uB = plsc.load_gather(uu_v, [pB])
                for k in range(16):
                    uBr = _lane_gather(uB, rot[k])
                    pBr = _lane_gather(pB, rot[k])
                    w = (uBr > uA) | ((uBr == uA) & (pBr < pA))
                    acc = acc + jnp.where(w, jnp.int32(1), jnp.int32(0))
                return acc

            rA = lax.fori_loop(0, nb2, rank_b, jnp.zeros((16,), jnp.int32))
            mk = rA < OUT_SZ
            widA = plsc.load_gather(ww_v, [pA])
            h0A = plsc.load_gather(hh_v, [pA])
            vA = plsc.bitcast(
                jnp.where(uA < 0, uA ^ jnp.int32(0x7FFFFFFF), uA),
                jnp.float32)
            plsc.store_scatter(oid_v, [rA], widA, mask=mk)
            plsc.store_scatter(opr_v, [2 * rA], h0A, mask=mk)
            plsc.store_scatter(opr_v, [2 * rA + 1], vA, mask=mk)
            return 0

        lax.fori_loop(0, nb2, rank_a, jnp.int32(0))
        pltpu.sync_copy(oid_v, ids_hbm.at[r0])
        pltpu.sync_copy(opr_v, pairs_hbm.at[r0])
        return 0

    lax.fori_loop(0, RPW, row_body, jnp.int32(0))


def kernel(x):
    b, c, hx, hy, h = x.shape
    x2d = x.reshape(ROWS, ROW_ELEMS)

    mesh = plsc.VectorSubcoreMesh(core_axis_name="c", subcore_axis_name="s",
                                  num_cores=NC, num_subcores=NS)
    ids, pairs = pl.kernel(
        _sc_kernel,
        out_type=[
            jax.ShapeDtypeStruct((ROWS, OX), jnp.int32),
            jax.ShapeDtypeStruct((ROWS, 2 * OX), jnp.float32),
        ],
        mesh=mesh,
        compiler_params=pltpu.CompilerParams(needs_layout_passes=False),
        scratch_types=[
            pltpu.VMEM((QB_ELEMS,), jnp.float32),
            pltpu.VMEM((NPOOL + 16,), jnp.int32),
            pltpu.VMEM((NPOOL + 16,), jnp.int32),
            pltpu.VMEM((NPOOL + 16,), jnp.float32),
            pltpu.VMEM((OX,), jnp.int32),
            pltpu.VMEM((_S1 + 16,), jnp.int32),
            pltpu.VMEM((_S1 + 16,), jnp.int32),
            pltpu.VMEM((_S2 + 16,), jnp.int32),
            pltpu.VMEM((OX,), jnp.int32),
            pltpu.VMEM((2 * OX,), jnp.float32),
        ],
    )(x2d)

    x_out = pairs.reshape(ROWS, OX, 2)[:, :OUT_SZ, :].reshape(
        b, c, OUT_SZ, 1, h)
    sorted_ids = ids[:, :OUT_SZ].reshape(b, c, OUT_SZ)
    return x_out, sorted_ids, hx, hy
